# baseline (speedup 1.0000x reference)
import functools
import math

import jax
import jax.numpy as jnp
from jax.experimental import pallas as pl
from jax.experimental.pallas import tpu as pltpu
from jax.experimental.pallas import tpu_sc as plsc

_HI = jax.lax.Precision.HIGHEST


def _stage_a(x_ref, wmu_ref, bmu_ref, wlv_ref, blv_ref, bits_ref):
    bs, d = x_ref.shape
    s = 2048
    x2 = x_ref[...]
    mu = jnp.dot(x2, wmu_ref[...], preferred_element_type=jnp.float32) + bmu_ref[...]
    lv = jnp.dot(x2, wlv_ref[...], preferred_element_type=jnp.float32) + blv_ref[...]
    lv = jnp.clip(lv, -8.0, 8.0)
    z = jnp.zeros((1, d), dtype=jnp.float32)
    mu_s = jnp.concatenate([z, mu[:-1]], axis=0)
    lv_s = jnp.concatenate([z, lv[:-1]], axis=0)
    scale = 0.5 / math.log(2.0)
    log_2pi = math.log(2.0 * math.pi)
    dx = x2 - mu_s
    t = jnp.square(dx) * jnp.exp(-lv_s) + lv_s + log_2pi
    bpd = jnp.mean(t, axis=-1, keepdims=True) * scale
    row = jax.lax.broadcasted_iota(jnp.int32, (bs, 1), 0)
    bits_ref[...] = jnp.where(row % s == 0, 0.0, bpd)


def _stage_b1(bits_ref, vf_ref, pe_ref, seg_ref, sc_ref):
    b, s = bits_ref.shape
    jc = 512
    bits = bits_ref[...]
    vf = vf_ref[...]
    sum_bits = jnp.sum(bits * vf)
    nv = jnp.sum(vf)
    nv0 = jnp.sum(vf[:, 0:1])
    thr = sum_bits / jnp.maximum(nv, 1.0)
    ent = sum_bits / jnp.maximum(nv - nv0, 1.0)
    colio = jax.lax.broadcasted_iota(jnp.int32, (b, s), 1)
    pe_bool = jnp.logical_and(bits > thr, vf > 0.0)
    pe = jnp.where(colio == s - 1, 1.0, jnp.where(pe_bool, 1.0, 0.0))
    pe_ref[...] = pe
    rio = jax.lax.broadcasted_iota(jnp.int32, (s, jc), 0)
    cio = jax.lax.broadcasted_iota(jnp.int32, (s, jc), 1)
    incs = []
    for c in range(s // jc):
        tri = (rio <= cio + c * jc).astype(jnp.float32)
        incs.append(jnp.dot(pe, tri, preferred_element_type=jnp.float32))
    inc = jnp.concatenate(incs, axis=1)
    seg_ref[0] = inc - pe
    io8 = jax.lax.broadcasted_iota(jnp.int32, (1, 8), 1)
    sc_ref[...] = jnp.where(io8 == 0, ent, jnp.where(io8 == 1, thr, 0.0))


def _stage_b2(seg_ref, x_ref, vfc_ref, pooled_ref, sv_ref, ns_ref):
    s, d = x_ref.shape
    jc = 512
    bb = pl.program_id(0)

    @pl.when(bb == 0)
    def _():
        ns_ref[...] = jnp.zeros_like(ns_ref)

    segb = seg_ref[0]
    vcb = vfc_ref[...]
    xm = x_ref[...] * vcb
    jio = jax.lax.broadcasted_iota(jnp.int32, (jc, 1), 0).astype(jnp.float32)
    cdn = (((1,), (0,)), ((), ()))
    nsegs = jnp.float32(0.0)
    for c in range(s // jc):
        a = (segb == (jio + c * jc)).astype(jnp.float32)
        sums = jax.lax.dot_general(a, xm, cdn, precision=_HI,
                                   preferred_element_type=jnp.float32)
        cnts = jax.lax.dot_general(a, vcb, cdn, precision=_HI,
                                   preferred_element_type=jnp.float32)
        pooled_ref[c * jc : (c + 1) * jc, :] = sums / jnp.maximum(cnts, 1.0)
        svc = (cnts > 0.0).astype(jnp.float32)
        sv_ref[c * jc : (c + 1) * jc, :] = svc
        nsegs = nsegs + jnp.sum(svc)
    io8 = jax.lax.broadcasted_iota(jnp.int32, (1, 8), 1)
    ns_ref[...] = ns_ref[...] + jnp.where(io8 == 0, nsegs, 0.0)


def _stage_c(pooled_ref, cbt_ref, sv_ref, idx_ref, cc_ref):
    rb, d = pooled_ref.shape
    k = cbt_ref.shape[1]
    kc = 2048
    nkc = k // kc
    r = pl.program_id(0)

    @pl.when(r == 0)
    def _():
        cc_ref[...] = jnp.zeros_like(cc_ref)

    cbt = cbt_ref[...]
    c2 = jnp.sum(cbt * cbt, axis=0, keepdims=True)
    p = pooled_ref[...]
    sv = sv_ref[...]
    p2 = jnp.sum(p * p, axis=1, keepdims=True)
    lio = jax.lax.broadcasted_iota(jnp.int32, (rb, kc), 1)
    cdn = (((1,), (0,)), ((), ()))
    best = jnp.full((rb, 1), jnp.inf, dtype=jnp.float32)
    bidx = jnp.zeros((rb, 1), dtype=jnp.int32)
    for c in range(nkc):
        pc = jax.lax.dot_general(p, cbt[:, c * kc : (c + 1) * kc], cdn,
                                 preferred_element_type=jnp.float32)
        d2 = p2 - 2.0 * pc + c2[:, c * kc : (c + 1) * kc]
        m = jnp.min(d2, axis=1, keepdims=True)
        li = jnp.min(jnp.where(d2 == m, lio + c * kc, k), axis=1, keepdims=True)
        upd = m < best
        bidx = jnp.where(upd, li, bidx)
        best = jnp.where(upd, m, best)
    idx_ref[...] = bidx
    for c in range(nkc):
        oh = (bidx == lio + c * kc).astype(jnp.float32)
        cc_ref[0:1, c * kc : (c + 1) * kc] = cc_ref[0:1, c * kc : (c + 1) * kc] + \
            jax.lax.dot_general(sv, oh, (((0,), (0,)), ((), ())),
                                preferred_element_type=jnp.float32)


def _stage_e(quant_ref, pooled_ref, sv_ref, cc_ref, sc_ref, vq_ref, out_ref):
    bs, d = pooled_ref.shape
    q = quant_ref[...]
    p = pooled_ref[...]
    sv = sv_ref[...]
    loss = jnp.sum(jnp.square(q - p) * sv)
    vq_ref[...] = (p + (q - p)) * sv
    ns = jnp.maximum(sc_ref[0, 0], 1.0)
    vq_loss = 1.25 * loss / (ns * d)
    probs = cc_ref[...] / ns
    perp = jnp.exp(-jnp.sum(probs * jnp.log(probs + 1e-10)))
    io8 = jax.lax.broadcasted_iota(jnp.int32, (1, 8), 1)
    out_ref[...] = jnp.where(io8 == 0, vq_loss, jnp.where(io8 == 1, perp, 0.0))


def _sc_gather(codebook, idx_flat):
    info = plsc.get_sparse_core_info()
    nc, ns_ = info.num_cores, info.num_subcores
    nw = nc * ns_
    bs = idx_flat.shape[0]
    d = codebook.shape[1]
    bpw = bs // nw
    mesh = plsc.VectorSubcoreMesh(core_axis_name="c", subcore_axis_name="s")

    @functools.partial(
        pl.kernel, mesh=mesh,
        out_type=jax.ShapeDtypeStruct((bs, d), jnp.float32),
        scratch_types=[
            pltpu.VMEM((bpw,), jnp.int32),
            pltpu.VMEM((bpw, d), jnp.float32),
            pltpu.SemaphoreType.DMA,
        ],
    )
    def k(table_hbm, idx_hbm, out_hbm, idx_v, rows_v, sem):
        wid = jax.lax.axis_index("s") * nc + jax.lax.axis_index("c")
        base = wid * bpw
        pltpu.sync_copy(idx_hbm.at[pl.ds(base, bpw)], idx_v)
        pltpu.async_copy(table_hbm.at[idx_v], rows_v, sem).wait()
        pltpu.sync_copy(rows_v, out_hbm.at[pl.ds(base, bpw)])

    return k(codebook, idx_flat)


def kernel(x, key_padding_mask, Wmu, bmu, Wlv, blv, codebook):
    b, s, d = x.shape
    k = codebook.shape[0]
    bs = b * s
    f32 = jnp.float32
    x2 = x.reshape(bs, d)
    vf = jnp.logical_not(key_padding_mask).astype(f32)
    vfc = vf.reshape(bs, 1)
    cbt = codebook.T

    bits_flat = pl.pallas_call(
        _stage_a,
        out_shape=jax.ShapeDtypeStruct((bs, 1), f32),
    )(x2, Wmu, bmu.reshape(1, d), Wlv, blv.reshape(1, d))

    bits_bs = bits_flat.reshape(b, s)
    pe_f, seg_id3, scal = pl.pallas_call(
        _stage_b1,
        out_shape=(
            jax.ShapeDtypeStruct((b, s), f32),
            jax.ShapeDtypeStruct((1, b, s), f32),
            jax.ShapeDtypeStruct((1, 8), f32),
        ),
    )(bits_bs, vf)

    seg_id3 = seg_id3.reshape(b, 1, s)
    pooled, sv, ns8 = pl.pallas_call(
        _stage_b2,
        grid=(b,),
        in_specs=[
            pl.BlockSpec((1, 1, s), lambda i: (i, 0, 0)),
            pl.BlockSpec((s, d), lambda i: (i, 0)),
            pl.BlockSpec((s, 1), lambda i: (i, 0)),
        ],
        out_specs=(
            pl.BlockSpec((s, d), lambda i: (i, 0)),
            pl.BlockSpec((s, 1), lambda i: (i, 0)),
            pl.BlockSpec((1, 8), lambda i: (0, 0)),
        ),
        out_shape=(
            jax.ShapeDtypeStruct((bs, d), f32),
            jax.ShapeDtypeStruct((bs, 1), f32),
            jax.ShapeDtypeStruct((1, 8), f32),
        ),
    )(seg_id3, x2, vfc)

    rb = 1024
    idx2, cc = pl.pallas_call(
        _stage_c,
        grid=(bs // rb,),
        in_specs=[
            pl.BlockSpec((rb, d), lambda i: (i, 0)),
            pl.BlockSpec((d, k), lambda i: (0, 0)),
            pl.BlockSpec((rb, 1), lambda i: (i, 0)),
        ],
        out_specs=(
            pl.BlockSpec((rb, 1), lambda i: (i, 0)),
            pl.BlockSpec((1, k), lambda i: (0, 0)),
        ),
        out_shape=(
            jax.ShapeDtypeStruct((bs, 1), jnp.int32),
            jax.ShapeDtypeStruct((1, k), f32),
        ),
    )(pooled, cbt, sv)

    idx = idx2.reshape(bs)
    cb_pad = jnp.pad(codebook, ((0, 0), (0, 128 - d)))
    quant = _sc_gather(cb_pad, idx)[:, :d]

    vq_emb, out2 = pl.pallas_call(
        _stage_e,
        out_shape=(
            jax.ShapeDtypeStruct((bs, d), f32),
            jax.ShapeDtypeStruct((1, 8), f32),
        ),
    )(quant, pooled, sv, cc, ns8)

    vq_loss = out2[0, 0]
    perplexity = out2[0, 1]
    entropy_loss = scal[0, 0]
    patch_end = pe_f > 0.5
    return vq_emb, idx, vq_loss, perplexity, bits_bs, entropy_loss, patch_end


# speedup vs baseline: 1.1023x; 1.1023x over previous
import functools
import math

import jax
import jax.numpy as jnp
from jax.experimental import pallas as pl
from jax.experimental.pallas import tpu as pltpu
from jax.experimental.pallas import tpu_sc as plsc

_HI = jax.lax.Precision.HIGHEST


def _stage_a(x_ref, wmu_ref, bmu_ref, wlv_ref, blv_ref, bits_ref):
    bs, d = x_ref.shape
    s = 2048
    x2 = x_ref[...]
    mu = jnp.dot(x2, wmu_ref[...], preferred_element_type=jnp.float32) + bmu_ref[...]
    lv = jnp.dot(x2, wlv_ref[...], preferred_element_type=jnp.float32) + blv_ref[...]
    lv = jnp.clip(lv, -8.0, 8.0)
    z = jnp.zeros((1, d), dtype=jnp.float32)
    mu_s = jnp.concatenate([z, mu[:-1]], axis=0)
    lv_s = jnp.concatenate([z, lv[:-1]], axis=0)
    scale = 0.5 / math.log(2.0)
    log_2pi = math.log(2.0 * math.pi)
    dx = x2 - mu_s
    t = jnp.square(dx) * jnp.exp(-lv_s) + lv_s + log_2pi
    bpd = jnp.mean(t, axis=-1, keepdims=True) * scale
    row = jax.lax.broadcasted_iota(jnp.int32, (bs, 1), 0)
    bits_ref[...] = jnp.where(row % s == 0, 0.0, bpd)


def _stage_b1(bits_ref, vf_ref, pe_ref, seg_ref, sc_ref):
    b, s = bits_ref.shape
    jc = 512
    bits = bits_ref[...]
    vf = vf_ref[...]
    sum_bits = jnp.sum(bits * vf)
    nv = jnp.sum(vf)
    nv0 = jnp.sum(vf[:, 0:1])
    thr = sum_bits / jnp.maximum(nv, 1.0)
    ent = sum_bits / jnp.maximum(nv - nv0, 1.0)
    colio = jax.lax.broadcasted_iota(jnp.int32, (b, s), 1)
    pe_bool = jnp.logical_and(bits > thr, vf > 0.0)
    pe = jnp.where(colio == s - 1, 1.0, jnp.where(pe_bool, 1.0, 0.0))
    pe_ref[...] = pe
    rio = jax.lax.broadcasted_iota(jnp.int32, (s, jc), 0)
    cio = jax.lax.broadcasted_iota(jnp.int32, (s, jc), 1)
    incs = []
    for c in range(s // jc):
        tri = (rio <= cio + c * jc).astype(jnp.float32)
        incs.append(jnp.dot(pe, tri, preferred_element_type=jnp.float32))
    inc = jnp.concatenate(incs, axis=1)
    seg_ref[0] = inc - pe
    io8 = jax.lax.broadcasted_iota(jnp.int32, (1, 8), 1)
    sc_ref[...] = jnp.where(io8 == 0, ent, jnp.where(io8 == 1, thr, 0.0))


def _stage_b2(seg_ref, x_ref, vfc_ref, pooled_ref, sv_ref, ns_ref):
    s, d = x_ref.shape
    jc = 512
    bb = pl.program_id(0)

    @pl.when(bb == 0)
    def _():
        ns_ref[...] = jnp.zeros_like(ns_ref)

    segb = seg_ref[0]
    vcb = vfc_ref[...]
    xm = x_ref[...] * vcb
    jio = jax.lax.broadcasted_iota(jnp.int32, (jc, 1), 0).astype(jnp.float32)
    cdn = (((1,), (0,)), ((), ()))
    nsegs = jnp.float32(0.0)
    for c in range(s // jc):
        a = (segb == (jio + c * jc)).astype(jnp.float32)
        sums = jax.lax.dot_general(a, xm, cdn, precision=_HI,
                                   preferred_element_type=jnp.float32)
        cnts = jax.lax.dot_general(a, vcb, cdn, precision=_HI,
                                   preferred_element_type=jnp.float32)
        pooled_ref[c * jc : (c + 1) * jc, :] = sums / jnp.maximum(cnts, 1.0)
        svc = (cnts > 0.0).astype(jnp.float32)
        sv_ref[c * jc : (c + 1) * jc, :] = svc
        nsegs = nsegs + jnp.sum(svc)
    io8 = jax.lax.broadcasted_iota(jnp.int32, (1, 8), 1)
    ns_ref[...] = ns_ref[...] + jnp.where(io8 == 0, nsegs, 0.0)


def _stage_c(pooled_ref, cbt_ref, sv_ref, idx_ref, cc_ref):
    rb, d = pooled_ref.shape
    k = cbt_ref.shape[1]
    kc = 2048
    nkc = k // kc
    r = pl.program_id(0)

    @pl.when(r == 0)
    def _():
        cc_ref[...] = jnp.zeros_like(cc_ref)

    cbt = cbt_ref[...]
    c2 = jnp.sum(cbt * cbt, axis=0, keepdims=True)
    p = pooled_ref[...]
    sv = sv_ref[...]
    lio = jax.lax.broadcasted_iota(jnp.int32, (rb, kc), 1)
    cdn = (((1,), (0,)), ((), ()))
    needed = jnp.max(sv) > 0.0

    @pl.when(needed)
    def _():
        p2 = jnp.sum(p * p, axis=1, keepdims=True)
        best = jnp.full((rb, 1), jnp.inf, dtype=jnp.float32)
        bidx = jnp.zeros((rb, 1), dtype=jnp.int32)
        for c in range(nkc):
            pc = jax.lax.dot_general(p, cbt[:, c * kc : (c + 1) * kc], cdn,
                                     preferred_element_type=jnp.float32)
            d2 = p2 - 2.0 * pc + c2[:, c * kc : (c + 1) * kc]
            m = jnp.min(d2, axis=1, keepdims=True)
            li = jnp.min(jnp.where(d2 == m, lio + c * kc, k), axis=1,
                         keepdims=True)
            upd = m < best
            bidx = jnp.where(upd, li, bidx)
            best = jnp.where(upd, m, best)
        idx_ref[...] = bidx
        for c in range(nkc):
            oh = (bidx == lio + c * kc).astype(jnp.float32)
            cc_ref[0:1, c * kc : (c + 1) * kc] = cc_ref[0:1, c * kc : (c + 1) * kc] + \
                jax.lax.dot_general(sv, oh, (((0,), (0,)), ((), ())),
                                    preferred_element_type=jnp.float32)

    @pl.when(jnp.logical_not(needed))
    def _():
        kio = jax.lax.broadcasted_iota(jnp.int32, (1, k), 1)
        m0 = jnp.min(c2, axis=1, keepdims=True)
        li0 = jnp.min(jnp.where(c2 == m0, kio, k), axis=1, keepdims=True)
        idx_ref[...] = jnp.broadcast_to(li0, (rb, 1))


def _stage_e(quant_ref, pooled_ref, sv_ref, cc_ref, sc_ref, vq_ref, out_ref):
    bs, d = pooled_ref.shape
    q = quant_ref[...]
    p = pooled_ref[...]
    sv = sv_ref[...]
    loss = jnp.sum(jnp.square(q - p) * sv)
    vq_ref[...] = (p + (q - p)) * sv
    ns = jnp.maximum(sc_ref[0, 0], 1.0)
    vq_loss = 1.25 * loss / (ns * d)
    probs = cc_ref[...] / ns
    perp = jnp.exp(-jnp.sum(probs * jnp.log(probs + 1e-10)))
    io8 = jax.lax.broadcasted_iota(jnp.int32, (1, 8), 1)
    out_ref[...] = jnp.where(io8 == 0, vq_loss, jnp.where(io8 == 1, perp, 0.0))


def _sc_gather(codebook, idx_flat):
    info = plsc.get_sparse_core_info()
    nc, ns_ = info.num_cores, info.num_subcores
    nw = nc * ns_
    bs = idx_flat.shape[0]
    d = codebook.shape[1]
    bpw = bs // nw
    mesh = plsc.VectorSubcoreMesh(core_axis_name="c", subcore_axis_name="s")

    @functools.partial(
        pl.kernel, mesh=mesh,
        out_type=jax.ShapeDtypeStruct((bs, d), jnp.float32),
        scratch_types=[
            pltpu.VMEM((bpw,), jnp.int32),
            pltpu.VMEM((bpw, d), jnp.float32),
            pltpu.SemaphoreType.DMA,
        ],
    )
    def k(table_hbm, idx_hbm, out_hbm, idx_v, rows_v, sem):
        wid = jax.lax.axis_index("s") * nc + jax.lax.axis_index("c")
        base = wid * bpw
        pltpu.sync_copy(idx_hbm.at[pl.ds(base, bpw)], idx_v)
        pltpu.async_copy(table_hbm.at[idx_v], rows_v, sem).wait()
        pltpu.sync_copy(rows_v, out_hbm.at[pl.ds(base, bpw)])

    return k(codebook, idx_flat)


def kernel(x, key_padding_mask, Wmu, bmu, Wlv, blv, codebook):
    b, s, d = x.shape
    k = codebook.shape[0]
    bs = b * s
    f32 = jnp.float32
    x2 = x.reshape(bs, d)
    vf = jnp.logical_not(key_padding_mask).astype(f32)
    vfc = vf.reshape(bs, 1)
    cbt = codebook.T

    bits_flat = pl.pallas_call(
        _stage_a,
        out_shape=jax.ShapeDtypeStruct((bs, 1), f32),
    )(x2, Wmu, bmu.reshape(1, d), Wlv, blv.reshape(1, d))

    bits_bs = bits_flat.reshape(b, s)
    pe_f, seg_id3, scal = pl.pallas_call(
        _stage_b1,
        out_shape=(
            jax.ShapeDtypeStruct((b, s), f32),
            jax.ShapeDtypeStruct((1, b, s), f32),
            jax.ShapeDtypeStruct((1, 8), f32),
        ),
    )(bits_bs, vf)

    seg_id3 = seg_id3.reshape(b, 1, s)
    pooled, sv, ns8 = pl.pallas_call(
        _stage_b2,
        grid=(b,),
        in_specs=[
            pl.BlockSpec((1, 1, s), lambda i: (i, 0, 0)),
            pl.BlockSpec((s, d), lambda i: (i, 0)),
            pl.BlockSpec((s, 1), lambda i: (i, 0)),
        ],
        out_specs=(
            pl.BlockSpec((s, d), lambda i: (i, 0)),
            pl.BlockSpec((s, 1), lambda i: (i, 0)),
            pl.BlockSpec((1, 8), lambda i: (0, 0)),
        ),
        out_shape=(
            jax.ShapeDtypeStruct((bs, d), f32),
            jax.ShapeDtypeStruct((bs, 1), f32),
            jax.ShapeDtypeStruct((1, 8), f32),
        ),
    )(seg_id3, x2, vfc)

    rb = 1024
    idx2, cc = pl.pallas_call(
        _stage_c,
        grid=(bs // rb,),
        in_specs=[
            pl.BlockSpec((rb, d), lambda i: (i, 0)),
            pl.BlockSpec((d, k), lambda i: (0, 0)),
            pl.BlockSpec((rb, 1), lambda i: (i, 0)),
        ],
        out_specs=(
            pl.BlockSpec((rb, 1), lambda i: (i, 0)),
            pl.BlockSpec((1, k), lambda i: (0, 0)),
        ),
        out_shape=(
            jax.ShapeDtypeStruct((bs, 1), jnp.int32),
            jax.ShapeDtypeStruct((1, k), f32),
        ),
    )(pooled, cbt, sv)

    idx = idx2.reshape(bs)
    cb_pad = jnp.pad(codebook, ((0, 0), (0, 128 - d)))
    quant = _sc_gather(cb_pad, idx)[:, :d]

    vq_emb, out2 = pl.pallas_call(
        _stage_e,
        out_shape=(
            jax.ShapeDtypeStruct((bs, d), f32),
            jax.ShapeDtypeStruct((1, 8), f32),
        ),
    )(quant, pooled, sv, cc, ns8)

    vq_loss = out2[0, 0]
    perplexity = out2[0, 1]
    entropy_loss = scal[0, 0]
    patch_end = pe_f > 0.5
    return vq_emb, idx, vq_loss, perplexity, bits_bs, entropy_loss, patch_end


# speedup vs baseline: 1.1025x; 1.0003x over previous
import functools
import math

import jax
import jax.numpy as jnp
from jax.experimental import pallas as pl
from jax.experimental.pallas import tpu as pltpu
from jax.experimental.pallas import tpu_sc as plsc

_HI = jax.lax.Precision.HIGHEST


def _stage_a(x_ref, wmu_ref, bmu_ref, wlv_ref, blv_ref, bits_ref):
    bs, d = x_ref.shape
    s = 2048
    x2 = x_ref[...]
    mu = jnp.dot(x2, wmu_ref[...], preferred_element_type=jnp.float32) + bmu_ref[...]
    lv = jnp.dot(x2, wlv_ref[...], preferred_element_type=jnp.float32) + blv_ref[...]
    lv = jnp.clip(lv, -8.0, 8.0)
    z = jnp.zeros((1, d), dtype=jnp.float32)
    mu_s = jnp.concatenate([z, mu[:-1]], axis=0)
    lv_s = jnp.concatenate([z, lv[:-1]], axis=0)
    scale = 0.5 / math.log(2.0)
    log_2pi = math.log(2.0 * math.pi)
    dx = x2 - mu_s
    t = jnp.square(dx) * jnp.exp(-lv_s) + lv_s + log_2pi
    bpd = jnp.mean(t, axis=-1, keepdims=True) * scale
    row = jax.lax.broadcasted_iota(jnp.int32, (bs, 1), 0)
    bits_ref[...] = jnp.where(row % s == 0, 0.0, bpd)


def _stage_b1(bits_ref, vf_ref, pe_ref, seg_ref, sc_ref):
    b, s = bits_ref.shape
    jc = 512
    bits = bits_ref[...]
    vf = vf_ref[...]
    sum_bits = jnp.sum(bits * vf)
    nv = jnp.sum(vf)
    nv0 = jnp.sum(vf[:, 0:1])
    thr = sum_bits / jnp.maximum(nv, 1.0)
    ent = sum_bits / jnp.maximum(nv - nv0, 1.0)
    colio = jax.lax.broadcasted_iota(jnp.int32, (b, s), 1)
    pe_bool = jnp.logical_and(bits > thr, vf > 0.0)
    pe = jnp.where(colio == s - 1, 1.0, jnp.where(pe_bool, 1.0, 0.0))
    pe_ref[...] = pe
    rio = jax.lax.broadcasted_iota(jnp.int32, (s, jc), 0)
    cio = jax.lax.broadcasted_iota(jnp.int32, (s, jc), 1)
    incs = []
    for c in range(s // jc):
        tri = (rio <= cio + c * jc).astype(jnp.float32)
        incs.append(jnp.dot(pe, tri, preferred_element_type=jnp.float32))
    inc = jnp.concatenate(incs, axis=1)
    seg_ref[0] = inc - pe
    io8 = jax.lax.broadcasted_iota(jnp.int32, (1, 8), 1)
    sc_ref[...] = jnp.where(io8 == 0, ent, jnp.where(io8 == 1, thr, 0.0))


def _stage_b2(seg_ref, x_ref, vfc_ref, pooled_ref, sv_ref, ns_ref):
    s, d = x_ref.shape
    jc = 512
    bb = pl.program_id(0)

    @pl.when(bb == 0)
    def _():
        ns_ref[...] = jnp.zeros_like(ns_ref)

    segb = seg_ref[0]
    vcb = vfc_ref[...]
    xm = x_ref[...] * vcb
    jio = jax.lax.broadcasted_iota(jnp.int32, (jc, 1), 0).astype(jnp.float32)
    cdn = (((1,), (0,)), ((), ()))
    nsegs = jnp.float32(0.0)
    for c in range(s // jc):
        a = (segb == (jio + c * jc)).astype(jnp.float32)
        sums = jax.lax.dot_general(a, xm, cdn, precision=_HI,
                                   preferred_element_type=jnp.float32)
        cnts = jax.lax.dot_general(a, vcb, cdn, precision=_HI,
                                   preferred_element_type=jnp.float32)
        pooled_ref[c * jc : (c + 1) * jc, :] = sums / jnp.maximum(cnts, 1.0)
        svc = (cnts > 0.0).astype(jnp.float32)
        sv_ref[c * jc : (c + 1) * jc, :] = svc
        nsegs = nsegs + jnp.sum(svc)
    io8 = jax.lax.broadcasted_iota(jnp.int32, (1, 8), 1)
    ns_ref[...] = ns_ref[...] + jnp.where(io8 == 0, nsegs, 0.0)


def _stage_c(pooled_ref, cbt_ref, sv_ref, idx_ref, cc_ref):
    rb, d = pooled_ref.shape
    k = cbt_ref.shape[1]
    kc = 2048
    nkc = k // kc
    r = pl.program_id(0)

    @pl.when(r == 0)
    def _():
        cc_ref[...] = jnp.zeros_like(cc_ref)

    cbt = cbt_ref[...]
    c2 = jnp.sum(cbt * cbt, axis=0, keepdims=True)
    p = pooled_ref[...]
    sv = sv_ref[...]
    lio = jax.lax.broadcasted_iota(jnp.int32, (rb, kc), 1)
    cdn = (((1,), (0,)), ((), ()))
    needed = jnp.max(sv) > 0.0

    @pl.when(needed)
    def _():
        p2 = jnp.sum(p * p, axis=1, keepdims=True)
        best = jnp.full((rb, 1), jnp.inf, dtype=jnp.float32)
        bidx = jnp.zeros((rb, 1), dtype=jnp.int32)
        for c in range(nkc):
            pc = jax.lax.dot_general(p, cbt[:, c * kc : (c + 1) * kc], cdn,
                                     preferred_element_type=jnp.float32)
            d2 = p2 - 2.0 * pc + c2[:, c * kc : (c + 1) * kc]
            m = jnp.min(d2, axis=1, keepdims=True)
            li = jnp.min(jnp.where(d2 == m, lio + c * kc, k), axis=1,
                         keepdims=True)
            upd = m < best
            bidx = jnp.where(upd, li, bidx)
            best = jnp.where(upd, m, best)
        idx_ref[...] = bidx
        for c in range(nkc):
            oh = (bidx == lio + c * kc).astype(jnp.float32)
            cc_ref[0:1, c * kc : (c + 1) * kc] = cc_ref[0:1, c * kc : (c + 1) * kc] + \
                jax.lax.dot_general(sv, oh, (((0,), (0,)), ((), ())),
                                    preferred_element_type=jnp.float32)

    @pl.when(jnp.logical_not(needed))
    def _():
        kio = jax.lax.broadcasted_iota(jnp.int32, (1, k), 1)
        m0 = jnp.min(c2, axis=1, keepdims=True)
        li0 = jnp.min(jnp.where(c2 == m0, kio, k), axis=1, keepdims=True)
        idx_ref[...] = jnp.broadcast_to(li0, (rb, 1))


def _stage_e(quant_ref, pooled_ref, sv_ref, cc_ref, sc_ref, vq_ref, out_ref):
    bs, d = pooled_ref.shape
    q = quant_ref[...]
    p = pooled_ref[...]
    sv = sv_ref[...]
    loss = jnp.sum(jnp.square(q - p) * sv)
    vq_ref[...] = (p + (q - p)) * sv
    ns = jnp.maximum(sc_ref[0, 0], 1.0)
    vq_loss = 1.25 * loss / (ns * d)
    probs = cc_ref[...] / ns
    perp = jnp.exp(-jnp.sum(probs * jnp.log(probs + 1e-10)))
    io8 = jax.lax.broadcasted_iota(jnp.int32, (1, 8), 1)
    out_ref[...] = jnp.where(io8 == 0, vq_loss, jnp.where(io8 == 1, perp, 0.0))


def _sc_gather(codebook, idx_flat):
    info = plsc.get_sparse_core_info()
    nc, ns_ = info.num_cores, info.num_subcores
    nw = nc * ns_
    bs = idx_flat.shape[0]
    d = codebook.shape[1]
    bpw = bs // nw
    mesh = plsc.VectorSubcoreMesh(core_axis_name="c", subcore_axis_name="s")

    @functools.partial(
        pl.kernel, mesh=mesh,
        out_type=jax.ShapeDtypeStruct((bs, d), jnp.float32),
        scratch_types=[
            pltpu.VMEM((bpw,), jnp.int32),
            pltpu.VMEM((bpw, d), jnp.float32),
            pltpu.SemaphoreType.DMA,
        ],
    )
    def k(table_hbm, idx_hbm, out_hbm, idx_v, rows_v, sem):
        wid = jax.lax.axis_index("s") * nc + jax.lax.axis_index("c")
        base = wid * bpw
        pltpu.sync_copy(idx_hbm.at[pl.ds(base, bpw)], idx_v)
        ch = 64
        cops = []
        for j in range(bpw // ch):
            cops.append(pltpu.async_copy(
                table_hbm.at[idx_v.at[pl.ds(j * ch, ch)]],
                rows_v.at[pl.ds(j * ch, ch)], sem))
        for cop in cops:
            cop.wait()
        pltpu.sync_copy(rows_v, out_hbm.at[pl.ds(base, bpw)])

    return k(codebook, idx_flat)


def kernel(x, key_padding_mask, Wmu, bmu, Wlv, blv, codebook):
    b, s, d = x.shape
    k = codebook.shape[0]
    bs = b * s
    f32 = jnp.float32
    x2 = x.reshape(bs, d)
    vf = jnp.logical_not(key_padding_mask).astype(f32)
    vfc = vf.reshape(bs, 1)
    cbt = codebook.T

    bits_flat = pl.pallas_call(
        _stage_a,
        out_shape=jax.ShapeDtypeStruct((bs, 1), f32),
    )(x2, Wmu, bmu.reshape(1, d), Wlv, blv.reshape(1, d))

    bits_bs = bits_flat.reshape(b, s)
    pe_f, seg_id3, scal = pl.pallas_call(
        _stage_b1,
        out_shape=(
            jax.ShapeDtypeStruct((b, s), f32),
            jax.ShapeDtypeStruct((1, b, s), f32),
            jax.ShapeDtypeStruct((1, 8), f32),
        ),
    )(bits_bs, vf)

    seg_id3 = seg_id3.reshape(b, 1, s)
    pooled, sv, ns8 = pl.pallas_call(
        _stage_b2,
        grid=(b,),
        in_specs=[
            pl.BlockSpec((1, 1, s), lambda i: (i, 0, 0)),
            pl.BlockSpec((s, d), lambda i: (i, 0)),
            pl.BlockSpec((s, 1), lambda i: (i, 0)),
        ],
        out_specs=(
            pl.BlockSpec((s, d), lambda i: (i, 0)),
            pl.BlockSpec((s, 1), lambda i: (i, 0)),
            pl.BlockSpec((1, 8), lambda i: (0, 0)),
        ),
        out_shape=(
            jax.ShapeDtypeStruct((bs, d), f32),
            jax.ShapeDtypeStruct((bs, 1), f32),
            jax.ShapeDtypeStruct((1, 8), f32),
        ),
    )(seg_id3, x2, vfc)

    rb = 1024
    idx2, cc = pl.pallas_call(
        _stage_c,
        grid=(bs // rb,),
        in_specs=[
            pl.BlockSpec((rb, d), lambda i: (i, 0)),
            pl.BlockSpec((d, k), lambda i: (0, 0)),
            pl.BlockSpec((rb, 1), lambda i: (i, 0)),
        ],
        out_specs=(
            pl.BlockSpec((rb, 1), lambda i: (i, 0)),
            pl.BlockSpec((1, k), lambda i: (0, 0)),
        ),
        out_shape=(
            jax.ShapeDtypeStruct((bs, 1), jnp.int32),
            jax.ShapeDtypeStruct((1, k), f32),
        ),
    )(pooled, cbt, sv)

    idx = idx2.reshape(bs)
    cb_pad = jnp.pad(codebook, ((0, 0), (0, 128 - d)))
    quant = _sc_gather(cb_pad, idx)[:, :d]

    vq_emb, out2 = pl.pallas_call(
        _stage_e,
        out_shape=(
            jax.ShapeDtypeStruct((bs, d), f32),
            jax.ShapeDtypeStruct((1, 8), f32),
        ),
    )(quant, pooled, sv, cc, ns8)

    vq_loss = out2[0, 0]
    perplexity = out2[0, 1]
    entropy_loss = scal[0, 0]
    patch_end = pe_f > 0.5
    return vq_emb, idx, vq_loss, perplexity, bits_bs, entropy_loss, patch_end


# speedup vs baseline: 1.4300x; 1.2970x over previous
import functools
import math

import jax
import jax.numpy as jnp
from jax.experimental import pallas as pl
from jax.experimental.pallas import tpu as pltpu
from jax.experimental.pallas import tpu_sc as plsc

_HI = jax.lax.Precision.HIGHEST


def _stage_a(x_ref, wmu_ref, bmu_ref, wlv_ref, blv_ref, bits_ref):
    bs, d = x_ref.shape
    s = 2048
    x2 = x_ref[...]
    mu = jnp.dot(x2, wmu_ref[...], preferred_element_type=jnp.float32) + bmu_ref[...]
    lv = jnp.dot(x2, wlv_ref[...], preferred_element_type=jnp.float32) + blv_ref[...]
    lv = jnp.clip(lv, -8.0, 8.0)
    z = jnp.zeros((1, d), dtype=jnp.float32)
    mu_s = jnp.concatenate([z, mu[:-1]], axis=0)
    lv_s = jnp.concatenate([z, lv[:-1]], axis=0)
    scale = 0.5 / math.log(2.0)
    log_2pi = math.log(2.0 * math.pi)
    dx = x2 - mu_s
    t = jnp.square(dx) * jnp.exp(-lv_s) + lv_s + log_2pi
    bpd = jnp.mean(t, axis=-1, keepdims=True) * scale
    row = jax.lax.broadcasted_iota(jnp.int32, (bs, 1), 0)
    bits_ref[...] = jnp.where(row % s == 0, 0.0, bpd)


def _stage_b1(bits_ref, vf_ref, pe_ref, seg_ref, sc_ref):
    b, s = bits_ref.shape
    jc = 512
    bits = bits_ref[...]
    vf = vf_ref[...]
    sum_bits = jnp.sum(bits * vf)
    nv = jnp.sum(vf)
    nv0 = jnp.sum(vf[:, 0:1])
    thr = sum_bits / jnp.maximum(nv, 1.0)
    ent = sum_bits / jnp.maximum(nv - nv0, 1.0)
    colio = jax.lax.broadcasted_iota(jnp.int32, (b, s), 1)
    pe_bool = jnp.logical_and(bits > thr, vf > 0.0)
    pe = jnp.where(colio == s - 1, 1.0, jnp.where(pe_bool, 1.0, 0.0))
    pe_ref[...] = pe
    rio = jax.lax.broadcasted_iota(jnp.int32, (s, jc), 0)
    cio = jax.lax.broadcasted_iota(jnp.int32, (s, jc), 1)
    incs = []
    for c in range(s // jc):
        tri = (rio <= cio + c * jc).astype(jnp.float32)
        incs.append(jnp.dot(pe, tri, preferred_element_type=jnp.float32))
    inc = jnp.concatenate(incs, axis=1)
    seg_ref[0] = inc - pe
    io8 = jax.lax.broadcasted_iota(jnp.int32, (1, 8), 1)
    sc_ref[...] = jnp.where(io8 == 0, ent, jnp.where(io8 == 1, thr, 0.0))


def _stage_b2(seg_ref, x_ref, vfc_ref, pooled_ref, sv_ref, ns_ref):
    s, d = x_ref.shape
    jc = 512
    bb = pl.program_id(0)

    @pl.when(bb == 0)
    def _():
        ns_ref[...] = jnp.zeros_like(ns_ref)

    segb = seg_ref[0]
    vcb = vfc_ref[...]
    xm = x_ref[...] * vcb
    jio = jax.lax.broadcasted_iota(jnp.int32, (jc, 1), 0).astype(jnp.float32)
    cdn = (((1,), (0,)), ((), ()))
    nsegs = jnp.float32(0.0)
    for c in range(s // jc):
        a = (segb == (jio + c * jc)).astype(jnp.float32)
        sums = jax.lax.dot_general(a, xm, cdn, precision=_HI,
                                   preferred_element_type=jnp.float32)
        cnts = jax.lax.dot_general(a, vcb, cdn, precision=_HI,
                                   preferred_element_type=jnp.float32)
        pooled_ref[c * jc : (c + 1) * jc, :] = sums / jnp.maximum(cnts, 1.0)
        svc = (cnts > 0.0).astype(jnp.float32)
        sv_ref[c * jc : (c + 1) * jc, :] = svc
        nsegs = nsegs + jnp.sum(svc)
    io8 = jax.lax.broadcasted_iota(jnp.int32, (1, 8), 1)
    ns_ref[...] = ns_ref[...] + jnp.where(io8 == 0, nsegs, 0.0)


def _stage_c(pooled_ref, cbt_ref, sv_ref, idx_ref, cc_ref):
    rb, d = pooled_ref.shape
    k = cbt_ref.shape[1]
    kc = 2048
    nkc = k // kc
    r = pl.program_id(0)

    @pl.when(r == 0)
    def _():
        cc_ref[...] = jnp.zeros_like(cc_ref)

    cbt = cbt_ref[...]
    c2 = jnp.sum(cbt * cbt, axis=0, keepdims=True)
    p = pooled_ref[...]
    sv = sv_ref[...]
    lio = jax.lax.broadcasted_iota(jnp.int32, (rb, kc), 1)
    cdn = (((1,), (0,)), ((), ()))
    needed = jnp.max(sv) > 0.0

    @pl.when(needed)
    def _():
        p2 = jnp.sum(p * p, axis=1, keepdims=True)
        best = jnp.full((rb, 1), jnp.inf, dtype=jnp.float32)
        bidx = jnp.zeros((rb, 1), dtype=jnp.int32)
        for c in range(nkc):
            pc = jax.lax.dot_general(p, cbt[:, c * kc : (c + 1) * kc], cdn,
                                     preferred_element_type=jnp.float32)
            d2 = p2 - 2.0 * pc + c2[:, c * kc : (c + 1) * kc]
            m = jnp.min(d2, axis=1, keepdims=True)
            li = jnp.min(jnp.where(d2 == m, lio + c * kc, k), axis=1,
                         keepdims=True)
            upd = m < best
            bidx = jnp.where(upd, li, bidx)
            best = jnp.where(upd, m, best)
        idx_ref[...] = bidx
        for c in range(nkc):
            oh = (bidx == lio + c * kc).astype(jnp.float32)
            cc_ref[0:1, c * kc : (c + 1) * kc] = cc_ref[0:1, c * kc : (c + 1) * kc] + \
                jax.lax.dot_general(sv, oh, (((0,), (0,)), ((), ())),
                                    preferred_element_type=jnp.float32)

    @pl.when(jnp.logical_not(needed))
    def _():
        kio = jax.lax.broadcasted_iota(jnp.int32, (1, k), 1)
        m0 = jnp.min(c2, axis=1, keepdims=True)
        li0 = jnp.min(jnp.where(c2 == m0, kio, k), axis=1, keepdims=True)
        idx_ref[...] = jnp.broadcast_to(li0, (rb, 1))


def _stage_e(quant_ref, pooled_ref, sv_ref, cc_ref, sc_ref, vq_ref, out_ref):
    bs, d = pooled_ref.shape
    q = quant_ref[...]
    p = pooled_ref[...]
    sv = sv_ref[...]
    loss = jnp.sum(jnp.square(q - p) * sv)
    vq_ref[...] = (p + (q - p)) * sv
    ns = jnp.maximum(sc_ref[0, 0], 1.0)
    vq_loss = 1.25 * loss / (ns * d)
    probs = cc_ref[...] / ns
    perp = jnp.exp(-jnp.sum(probs * jnp.log(probs + 1e-10)))
    io8 = jax.lax.broadcasted_iota(jnp.int32, (1, 8), 1)
    out_ref[...] = jnp.where(io8 == 0, vq_loss, jnp.where(io8 == 1, perp, 0.0))


def _sc_gather(codebook, idx_flat):
    info = plsc.get_sparse_core_info()
    nc, ns_ = info.num_cores, info.num_subcores
    nw = nc * ns_
    bs = idx_flat.shape[0]
    d = codebook.shape[1]
    bpw = bs // nw
    mesh = plsc.VectorSubcoreMesh(core_axis_name="c", subcore_axis_name="s")

    @functools.partial(
        pl.kernel, mesh=mesh,
        compiler_params=pltpu.CompilerParams(use_tc_tiling_on_sc=False),
        out_type=jax.ShapeDtypeStruct((bs, d), jnp.float32),
        scratch_types=[
            pltpu.VMEM((bpw,), jnp.int32),
            pltpu.VMEM((bpw, d), jnp.float32),
            pltpu.SemaphoreType.DMA,
        ],
    )
    def k(table_hbm, idx_hbm, out_hbm, idx_v, rows_v, sem):
        wid = jax.lax.axis_index("s") * nc + jax.lax.axis_index("c")
        base = wid * bpw
        pltpu.sync_copy(idx_hbm.at[pl.ds(base, bpw)], idx_v)
        ch = 64
        cops = []
        for j in range(bpw // ch):
            cops.append(pltpu.async_copy(
                table_hbm.at[idx_v.at[pl.ds(j * ch, ch)]],
                rows_v.at[pl.ds(j * ch, ch)], sem))
        for cop in cops:
            cop.wait()
        pltpu.sync_copy(rows_v, out_hbm.at[pl.ds(base, bpw)])

    return k(codebook, idx_flat)


def kernel(x, key_padding_mask, Wmu, bmu, Wlv, blv, codebook):
    b, s, d = x.shape
    k = codebook.shape[0]
    bs = b * s
    f32 = jnp.float32
    x2 = x.reshape(bs, d)
    vf = jnp.logical_not(key_padding_mask).astype(f32)
    vfc = vf.reshape(bs, 1)
    cbt = codebook.T

    bits_flat = pl.pallas_call(
        _stage_a,
        out_shape=jax.ShapeDtypeStruct((bs, 1), f32),
    )(x2, Wmu, bmu.reshape(1, d), Wlv, blv.reshape(1, d))

    bits_bs = bits_flat.reshape(b, s)
    pe_f, seg_id3, scal = pl.pallas_call(
        _stage_b1,
        out_shape=(
            jax.ShapeDtypeStruct((b, s), f32),
            jax.ShapeDtypeStruct((1, b, s), f32),
            jax.ShapeDtypeStruct((1, 8), f32),
        ),
    )(bits_bs, vf)

    seg_id3 = seg_id3.reshape(b, 1, s)
    pooled, sv, ns8 = pl.pallas_call(
        _stage_b2,
        grid=(b,),
        in_specs=[
            pl.BlockSpec((1, 1, s), lambda i: (i, 0, 0)),
            pl.BlockSpec((s, d), lambda i: (i, 0)),
            pl.BlockSpec((s, 1), lambda i: (i, 0)),
        ],
        out_specs=(
            pl.BlockSpec((s, d), lambda i: (i, 0)),
            pl.BlockSpec((s, 1), lambda i: (i, 0)),
            pl.BlockSpec((1, 8), lambda i: (0, 0)),
        ),
        out_shape=(
            jax.ShapeDtypeStruct((bs, d), f32),
            jax.ShapeDtypeStruct((bs, 1), f32),
            jax.ShapeDtypeStruct((1, 8), f32),
        ),
    )(seg_id3, x2, vfc)

    rb = 1024
    idx2, cc = pl.pallas_call(
        _stage_c,
        grid=(bs // rb,),
        in_specs=[
            pl.BlockSpec((rb, d), lambda i: (i, 0)),
            pl.BlockSpec((d, k), lambda i: (0, 0)),
            pl.BlockSpec((rb, 1), lambda i: (i, 0)),
        ],
        out_specs=(
            pl.BlockSpec((rb, 1), lambda i: (i, 0)),
            pl.BlockSpec((1, k), lambda i: (0, 0)),
        ),
        out_shape=(
            jax.ShapeDtypeStruct((bs, 1), jnp.int32),
            jax.ShapeDtypeStruct((1, k), f32),
        ),
    )(pooled, cbt, sv)

    idx = idx2.reshape(bs)
    quant = _sc_gather(codebook, idx)

    vq_emb, out2 = pl.pallas_call(
        _stage_e,
        out_shape=(
            jax.ShapeDtypeStruct((bs, d), f32),
            jax.ShapeDtypeStruct((1, 8), f32),
        ),
    )(quant, pooled, sv, cc, ns8)

    vq_loss = out2[0, 0]
    perplexity = out2[0, 1]
    entropy_loss = scal[0, 0]
    patch_end = pe_f > 0.5
    return vq_emb, idx, vq_loss, perplexity, bits_bs, entropy_loss, patch_end
